# Initial kernel scaffold; baseline (speedup 1.0000x reference)
import jax
import jax.numpy as jnp
from jax.experimental import pallas as pl


def kernel(attr_item, attr_tf_item, attr_lens_item, item_ids, attr_user, attr_tf_user, attr_lens_user, user_ids, pos_targets, pos_lens, neg_targets, neg_lens, user_table, item_table, out_attr_user_table, out_attr_item_table):
    raise NotImplementedError("write your pallas kernel here")


# speedup vs baseline: 8.8341x; 8.8341x over previous
import functools

import jax
import jax.numpy as jnp
from jax import lax
from jax.experimental import pallas as pl
from jax.experimental.pallas import tpu as pltpu
from jax.experimental.pallas import tpu_sc as plsc

B = 16384
LP = 20
LN = 200
D = 64
TPAD = 224
CHUNK = 112
W = 32
RPW = B // W
RB = 64
NBLK = RPW // RB

_info = plsc.get_sparse_core_info()
_NC = _info.num_cores


def _sc_logits(tgt3, user_ids, item_ids, user_table, item_table, aut, ait):
  mesh = plsc.VectorSubcoreMesh(core_axis_name="c", subcore_axis_name="s")

  @functools.partial(
      pl.kernel,
      mesh=mesh,
      compiler_params=pltpu.CompilerParams(
          needs_layout_passes=False, use_tc_tiling_on_sc=False),
      out_type=jax.ShapeDtypeStruct((B, TPAD), jnp.float32),
      scratch_types=[
          pltpu.VMEM((RB, 2, CHUNK), jnp.int32),
          pltpu.VMEM((RB,), jnp.int32),
          pltpu.VMEM((RB,), jnp.int32),
          pltpu.VMEM((RB, D), jnp.float32),
          pltpu.VMEM((RB, D), jnp.float32),
          pltpu.VMEM((TPAD, D), jnp.float32),
          pltpu.VMEM((TPAD, D), jnp.float32),
          pltpu.VMEM((RB, TPAD), jnp.float32),
          pltpu.SemaphoreType.DMA,
      ],
  )
  def k(tgt_hbm, uid_hbm, iid_hbm, ut_hbm, it_hbm, aut_hbm, ait_hbm,
        out_hbm, tidx, uid, iid, urows, vrows, tu, ti, lbuf, sem):
    wid = lax.axis_index("s") * _NC + lax.axis_index("c")

    def do_block(blk, _):
      base = wid * RPW + blk * RB
      pltpu.sync_copy(tgt_hbm.at[pl.ds(base, RB)], tidx)
      pltpu.sync_copy(uid_hbm.at[pl.ds(base, RB)], uid)
      pltpu.sync_copy(iid_hbm.at[pl.ds(base, RB)], iid)
      pltpu.async_copy(ut_hbm.at[uid], urows, sem).wait()
      pltpu.async_copy(it_hbm.at[iid], vrows, sem).wait()

      iota16 = lax.broadcasted_iota(jnp.int32, (16,), 0)
      ngrp = TPAD // 16

      def do_row(r, _):
        cps = [
            pltpu.async_copy(aut_hbm.at[tidx.at[r, 0]],
                             tu.at[pl.ds(0, CHUNK)], sem),
            pltpu.async_copy(aut_hbm.at[tidx.at[r, 1]],
                             tu.at[pl.ds(CHUNK, CHUNK)], sem),
            pltpu.async_copy(ait_hbm.at[tidx.at[r, 0]],
                             ti.at[pl.ds(0, CHUNK)], sem),
            pltpu.async_copy(ait_hbm.at[tidx.at[r, 1]],
                             ti.at[pl.ds(CHUNK, CHUNK)], sem),
        ]
        for cp in cps:
          cp.wait()

        uj = [urows[r, pl.ds(16 * j, 16)] for j in range(4)]
        vj = [vrows[r, pl.ds(16 * j, 16)] for j in range(4)]

        def do_grp(g, _):
          def do_l(l, res):
            t = g * 16 + l
            s = tu[t, pl.ds(0, 16)] * uj[0]
            for j in range(1, 4):
              s = s + tu[t, pl.ds(16 * j, 16)] * uj[j]
            for j in range(4):
              s = s + ti[t, pl.ds(16 * j, 16)] * vj[j]
            tot = jnp.sum(s)
            return jnp.where(iota16 == l, jnp.full((16,), tot), res)

          res = lax.fori_loop(0, 16, do_l, jnp.zeros((16,), jnp.float32))
          lbuf[r, pl.ds(g * 16, 16)] = res
          return 0

        lax.fori_loop(0, ngrp, do_grp, 0)
        return 0

      lax.fori_loop(0, RB, do_row, 0)
      pltpu.sync_copy(lbuf, out_hbm.at[pl.ds(base, RB)])
      return 0

    lax.fori_loop(0, NBLK, do_block, 0)

  return k(tgt3, user_ids, item_ids, user_table, item_table, aut, ait)


MB = 512


MW = 256


def _mask_body(pl_ref, nl_ref, mask_ref, nt_ref):
  col = lax.broadcasted_iota(jnp.int32, (MB, MW), 1)
  p = pl_ref[...]
  n = nl_ref[...]
  a = jnp.clip(p - col, 0, 1)
  b = jnp.clip(n - (col - LP), 0, 1)
  ip = jnp.clip(LP - col, 0, 1)
  mask_ref[...] = ip * a + (1 - ip) * b
  nt_ref[...] = ip * a


def _masks(pos_lens, neg_lens):
  pl2 = pos_lens.reshape(B, 1).astype(jnp.int32)
  nl2 = neg_lens.reshape(B, 1).astype(jnp.int32)
  return pl.pallas_call(
      _mask_body,
      grid=(B // MB,),
      in_specs=[pl.BlockSpec((MB, 1), lambda i: (i, 0)),
                pl.BlockSpec((MB, 1), lambda i: (i, 0))],
      out_specs=[pl.BlockSpec((MB, MW), lambda i: (i, 0)),
                 pl.BlockSpec((MB, MW), lambda i: (i, 0))],
      out_shape=[jax.ShapeDtypeStruct((B, MW), jnp.int32),
                 jax.ShapeDtypeStruct((B, MW), jnp.int32)],
  )(pl2, nl2)


def kernel(attr_item, attr_tf_item, attr_lens_item, item_ids, attr_user,
           attr_tf_user, attr_lens_user, user_ids, pos_targets, pos_lens,
           neg_targets, neg_lens, user_table, item_table,
           out_attr_user_table, out_attr_item_table):
  tgt = jnp.concatenate(
      [pos_targets.astype(jnp.int32), neg_targets.astype(jnp.int32),
       jnp.zeros((B, TPAD - LP - LN), jnp.int32)], axis=1).reshape(B, 2, CHUNK)
  logits_pad = _sc_logits(tgt, user_ids.astype(jnp.int32),
                          item_ids.astype(jnp.int32), user_table, item_table,
                          out_attr_user_table, out_attr_item_table)
  logits = logits_pad[:, :LP + LN]
  mask_pad, nt_pad = _masks(pos_lens, neg_lens)
  return (logits, mask_pad[:, :LP + LN] != 0, nt_pad[:, :LP + LN])
